# Initial kernel scaffold; baseline (speedup 1.0000x reference)
import jax
import jax.numpy as jnp
from jax.experimental import pallas as pl


def kernel(y_n, unique_cell_types):
    raise NotImplementedError("write your pallas kernel here")


# speedup vs baseline: 2018.7163x; 2018.7163x over previous
import functools

import jax
import jax.numpy as jnp
from jax import lax
from jax.experimental import pallas as pl
from jax.experimental.pallas import tpu as pltpu
from jax.experimental.pallas import tpu_sc as plsc

_L = 16
_NUM_WORKERS = 32


def _encode_body(u_hbm, y_hbm, out_hbm, u_v, tab_v, y_v, o_v):
    t = u_v.shape[0]
    per_w = y_v.shape[0]
    wid = lax.axis_index("s") * 2 + lax.axis_index("c")
    base = wid * per_w
    pltpu.sync_copy(u_hbm, u_v)
    pltpu.sync_copy(y_hbm.at[pl.ds(base, per_w)], y_v)

    zeros = jnp.zeros((_L,), jnp.int32)
    for c in range(t // _L):
        tab_v[pl.ds(c * _L, _L)] = zeros
    ones = jnp.ones((_L,), jnp.int32)
    for c in range(t // _L):
        plsc.addupdate_scatter(tab_v, [u_v[pl.ds(c * _L, _L)]], ones)

    carry = jnp.int32(0)
    for c in range(t // _L):
        h = tab_v[pl.ds(c * _L, _L)]
        incl = plsc.cumsum(h)
        tab_v[pl.ds(c * _L, _L)] = incl - h + carry
        carry = carry + jnp.sum(h)

    def gbody(i, acc):
        sl = pl.ds(i * _L, _L)
        o_v[sl] = plsc.load_gather(tab_v, [y_v[sl]])
        return acc

    lax.fori_loop(0, per_w // _L, gbody, 0)
    pltpu.sync_copy(o_v, out_hbm.at[pl.ds(base, per_w)])


def kernel(y_n, unique_cell_types):
    y = y_n.astype(jnp.int32)
    u = unique_cell_types.astype(jnp.int32)
    n = y.shape[0]
    t = u.shape[0]
    per_w = n // _NUM_WORKERS
    mesh = plsc.VectorSubcoreMesh(core_axis_name="c", subcore_axis_name="s")
    run = functools.partial(
        pl.kernel,
        mesh=mesh,
        compiler_params=pltpu.CompilerParams(needs_layout_passes=False),
        out_type=jax.ShapeDtypeStruct((n,), jnp.int32),
        scratch_types=[
            pltpu.VMEM((t,), jnp.int32),
            pltpu.VMEM((t,), jnp.int32),
            pltpu.VMEM((per_w,), jnp.int32),
            pltpu.VMEM((per_w,), jnp.int32),
        ],
    )(_encode_body)
    out = run(u, y)
    return (out, out)


# speedup vs baseline: 2644.7108x; 1.3101x over previous
import functools

import jax
import jax.numpy as jnp
from jax import lax
from jax.experimental import pallas as pl
from jax.experimental.pallas import tpu as pltpu
from jax.experimental.pallas import tpu_sc as plsc

_L = 16
_NUM_WORKERS = 32


def _encode_body(u_hbm, y_hbm, out_hbm, u_v, tab_v, y_v, o_v):
    t = u_v.shape[0]
    per_w = y_v.shape[0]
    wid = lax.axis_index("s") * 2 + lax.axis_index("c")
    base = wid * per_w
    pltpu.sync_copy(u_hbm, u_v)
    pltpu.sync_copy(y_hbm.at[pl.ds(base, per_w)], y_v)

    zeros = jnp.zeros((_L,), jnp.int32)
    for c in range(t // _L):
        tab_v[pl.ds(c * _L, _L)] = zeros
    ones = jnp.ones((_L,), jnp.int32)
    for c in range(t // _L):
        plsc.addupdate_scatter(tab_v, [u_v[pl.ds(c * _L, _L)]], ones)

    carry = jnp.int32(0)
    for c in range(t // _L):
        h = tab_v[pl.ds(c * _L, _L)]
        incl = plsc.cumsum(h)
        tab_v[pl.ds(c * _L, _L)] = incl - h + carry
        carry = carry + jnp.sum(h)

    @plsc.parallel_loop(0, per_w, _L, unroll=8)
    def _gbody(i):
        sl = pl.ds(i, _L)
        o_v[sl] = plsc.load_gather(tab_v, [y_v[sl]])
    pltpu.sync_copy(o_v, out_hbm.at[pl.ds(base, per_w)])


def kernel(y_n, unique_cell_types):
    y = y_n.astype(jnp.int32)
    u = unique_cell_types.astype(jnp.int32)
    n = y.shape[0]
    t = u.shape[0]
    per_w = n // _NUM_WORKERS
    mesh = plsc.VectorSubcoreMesh(core_axis_name="c", subcore_axis_name="s")
    run = functools.partial(
        pl.kernel,
        mesh=mesh,
        compiler_params=pltpu.CompilerParams(needs_layout_passes=False),
        out_type=jax.ShapeDtypeStruct((n,), jnp.int32),
        scratch_types=[
            pltpu.VMEM((t,), jnp.int32),
            pltpu.VMEM((t,), jnp.int32),
            pltpu.VMEM((per_w,), jnp.int32),
            pltpu.VMEM((per_w,), jnp.int32),
        ],
    )(_encode_body)
    out = run(u, y)
    return (out, out)


# speedup vs baseline: 2692.6201x; 1.0181x over previous
import functools

import jax
import jax.numpy as jnp
from jax import lax
from jax.experimental import pallas as pl
from jax.experimental.pallas import tpu as pltpu
from jax.experimental.pallas import tpu_sc as plsc

_L = 16
_NUM_WORKERS = 32


_CHUNK = 8192


def _encode_body(u_hbm, y_hbm, out_hbm, u_v, tab_v,
                 y_v0, y_v1, o_v0, o_v1, in_s0, in_s1, out_s0, out_s1):
    t = u_v.shape[0]
    per_w = _CHUNK * 4
    nchunks = per_w // _CHUNK
    y_bufs, o_bufs = (y_v0, y_v1), (o_v0, o_v1)
    in_sems, out_sems = (in_s0, in_s1), (out_s0, out_s1)
    wid = lax.axis_index("s") * 2 + lax.axis_index("c")
    base = wid * per_w

    in_copies = [None] * nchunks
    out_copies = [None] * nchunks
    in_copies[0] = pltpu.make_async_copy(
        y_hbm.at[pl.ds(base, _CHUNK)], y_bufs[0], in_sems[0])
    in_copies[0].start()
    pltpu.sync_copy(u_hbm, u_v)

    zeros = jnp.zeros((_L,), jnp.int32)
    for c in range(t // _L):
        tab_v[pl.ds(c * _L, _L)] = zeros
    ones = jnp.ones((_L,), jnp.int32)
    for c in range(t // _L):
        plsc.addupdate_scatter(tab_v, [u_v[pl.ds(c * _L, _L)]], ones)

    carry = jnp.int32(0)
    for c in range(t // _L):
        h = tab_v[pl.ds(c * _L, _L)]
        incl = plsc.cumsum(h)
        tab_v[pl.ds(c * _L, _L)] = incl - h + carry
        carry = carry + jnp.sum(h)

    for g in range(nchunks):
        b = g & 1
        in_copies[g].wait()
        if g + 1 < nchunks:
            in_copies[g + 1] = pltpu.make_async_copy(
                y_hbm.at[pl.ds(base + (g + 1) * _CHUNK, _CHUNK)],
                y_bufs[(g + 1) & 1], in_sems[(g + 1) & 1])
            in_copies[g + 1].start()
        if g >= 2:
            out_copies[g - 2].wait()

        y_v, o_v = y_bufs[b], o_bufs[b]

        @plsc.parallel_loop(0, _CHUNK, _L, unroll=8)
        def _gbody(i):
            sl = pl.ds(i, _L)
            o_v[sl] = plsc.load_gather(tab_v, [y_v[sl]])

        out_copies[g] = pltpu.make_async_copy(
            o_v, out_hbm.at[pl.ds(base + g * _CHUNK, _CHUNK)], out_sems[b])
        out_copies[g].start()
    out_copies[nchunks - 2].wait()
    out_copies[nchunks - 1].wait()


def kernel(y_n, unique_cell_types):
    y = y_n.astype(jnp.int32)
    u = unique_cell_types.astype(jnp.int32)
    n = y.shape[0]
    t = u.shape[0]
    per_w = n // _NUM_WORKERS
    assert per_w == _CHUNK * 4
    mesh = plsc.VectorSubcoreMesh(core_axis_name="c", subcore_axis_name="s")
    run = functools.partial(
        pl.kernel,
        mesh=mesh,
        compiler_params=pltpu.CompilerParams(needs_layout_passes=False),
        out_type=jax.ShapeDtypeStruct((n,), jnp.int32),
        scratch_types=[
            pltpu.VMEM((t,), jnp.int32),
            pltpu.VMEM((t,), jnp.int32),
            pltpu.VMEM((_CHUNK,), jnp.int32),
            pltpu.VMEM((_CHUNK,), jnp.int32),
            pltpu.VMEM((_CHUNK,), jnp.int32),
            pltpu.VMEM((_CHUNK,), jnp.int32),
            pltpu.SemaphoreType.DMA,
            pltpu.SemaphoreType.DMA,
            pltpu.SemaphoreType.DMA,
            pltpu.SemaphoreType.DMA,
        ],
    )(_encode_body)
    out = run(u, y)
    return (out, out)


# speedup vs baseline: 3026.1072x; 1.1239x over previous
import functools

import jax
import jax.numpy as jnp
from jax import lax
from jax.experimental import pallas as pl
from jax.experimental.pallas import tpu as pltpu
from jax.experimental.pallas import tpu_sc as plsc

_L = 16
_NUM_WORKERS = 32


_CHUNK = 8192


def _encode_body(u_hbm, y_hbm, out_hbm, out2_hbm, u_v, tab_v,
                 y_v0, y_v1, o_v0, o_v1, in_s0, in_s1, out_s0, out_s1,
                 out2_s0, out2_s1):
    t = u_v.shape[0]
    per_w = _CHUNK * 4
    nchunks = per_w // _CHUNK
    y_bufs, o_bufs = (y_v0, y_v1), (o_v0, o_v1)
    in_sems, out_sems = (in_s0, in_s1), (out_s0, out_s1)
    out2_sems = (out2_s0, out2_s1)
    wid = lax.axis_index("s") * 2 + lax.axis_index("c")
    base = wid * per_w

    in_copies = [None] * nchunks
    out_copies = [None] * nchunks
    out2_copies = [None] * nchunks
    in_copies[0] = pltpu.make_async_copy(
        y_hbm.at[pl.ds(base, _CHUNK)], y_bufs[0], in_sems[0])
    in_copies[0].start()
    pltpu.sync_copy(u_hbm, u_v)

    zeros = jnp.zeros((_L,), jnp.int32)
    for c in range(t // _L):
        tab_v[pl.ds(c * _L, _L)] = zeros
    ones = jnp.ones((_L,), jnp.int32)
    for c in range(t // _L):
        plsc.addupdate_scatter(tab_v, [u_v[pl.ds(c * _L, _L)]], ones)

    carry = jnp.int32(0)
    for c in range(t // _L):
        h = tab_v[pl.ds(c * _L, _L)]
        incl = plsc.cumsum(h)
        tab_v[pl.ds(c * _L, _L)] = incl - h + carry
        carry = carry + jnp.sum(h)

    for g in range(nchunks):
        b = g & 1
        in_copies[g].wait()
        if g + 1 < nchunks:
            in_copies[g + 1] = pltpu.make_async_copy(
                y_hbm.at[pl.ds(base + (g + 1) * _CHUNK, _CHUNK)],
                y_bufs[(g + 1) & 1], in_sems[(g + 1) & 1])
            in_copies[g + 1].start()
        if g >= 2:
            out_copies[g - 2].wait()
            out2_copies[g - 2].wait()

        y_v, o_v = y_bufs[b], o_bufs[b]

        @plsc.parallel_loop(0, _CHUNK, _L, unroll=8)
        def _gbody(i):
            sl = pl.ds(i, _L)
            o_v[sl] = plsc.load_gather(tab_v, [y_v[sl]])

        out_copies[g] = pltpu.make_async_copy(
            o_v, out_hbm.at[pl.ds(base + g * _CHUNK, _CHUNK)], out_sems[b])
        out_copies[g].start()
        out2_copies[g] = pltpu.make_async_copy(
            o_v, out2_hbm.at[pl.ds(base + g * _CHUNK, _CHUNK)], out2_sems[b])
        out2_copies[g].start()
    for g in (nchunks - 2, nchunks - 1):
        out_copies[g].wait()
        out2_copies[g].wait()


def kernel(y_n, unique_cell_types):
    y = y_n.astype(jnp.int32)
    u = unique_cell_types.astype(jnp.int32)
    n = y.shape[0]
    t = u.shape[0]
    per_w = n // _NUM_WORKERS
    assert per_w == _CHUNK * 4
    mesh = plsc.VectorSubcoreMesh(core_axis_name="c", subcore_axis_name="s")
    run = functools.partial(
        pl.kernel,
        mesh=mesh,
        compiler_params=pltpu.CompilerParams(needs_layout_passes=False),
        out_type=(jax.ShapeDtypeStruct((n,), jnp.int32),
                  jax.ShapeDtypeStruct((n,), jnp.int32)),
        scratch_types=[
            pltpu.VMEM((t,), jnp.int32),
            pltpu.VMEM((t,), jnp.int32),
            pltpu.VMEM((_CHUNK,), jnp.int32),
            pltpu.VMEM((_CHUNK,), jnp.int32),
            pltpu.VMEM((_CHUNK,), jnp.int32),
            pltpu.VMEM((_CHUNK,), jnp.int32),
            pltpu.SemaphoreType.DMA,
            pltpu.SemaphoreType.DMA,
            pltpu.SemaphoreType.DMA,
            pltpu.SemaphoreType.DMA,
            pltpu.SemaphoreType.DMA,
            pltpu.SemaphoreType.DMA,
        ],
    )(_encode_body)
    out, out2 = run(u, y)
    return (out, out2)
